# baseline (speedup 1.0000x reference)
import functools

import jax
import jax.numpy as jnp
from jax import lax
from jax.experimental import pallas as pl
from jax.experimental.pallas import tpu as pltpu
from jax.experimental.pallas import tpu_sc as plsc

CTX = 1000
B, S = 1024, 50
ROWS = B * S
NW = 32
RPW = ROWS // NW
ZCH = 80
NCH = RPW // ZCH
ZW = ZCH * CTX
SLICE = ZW // 16
L = 16
WIN = 4


def _body(seq_hbm, out_hbm, idx_v, idx2d, stage, ones_v, zeros_sh,
          semz, sems):
    cid = lax.axis_index("c")
    sid = lax.axis_index("s")
    wid = sid * 2 + cid
    row0 = wid * RPW

    pltpu.sync_copy(seq_hbm.at[pl.ds(row0, RPW)], idx_v)

    zero16 = jnp.zeros((L,), jnp.float32)
    one16 = jnp.full((L,), 1.0, jnp.float32)

    ZUNROLL = 16
    def _zero_body(i, carry):
        base = i * (ZUNROLL * L)
        for k in range(ZUNROLL):
            stage[pl.ds(base + k * L, L)] = zero16
        return carry
    lax.fori_loop(0, SLICE // (ZUNROLL * L) + 1, _zero_body, 0)
    pltpu.sync_copy(stage.at[pl.ds(0, SLICE)],
                    zeros_sh.at[pl.ds(sid * SLICE, SLICE)])
    for o in range(0, ZCH, L):
        ones_v[pl.ds(o, L)] = one16

    iota_ctx = lax.iota(jnp.int32, L) * CTX
    row0k = row0 * CTX
    for c in range(NCH):
        for o in range(0, ZCH, L):
            g16 = idx_v[pl.ds(c * ZCH + o, L)]
            idx2d[c, pl.ds(o, L)] = g16 + iota_ctx + (row0k + (c * ZCH + o) * CTX)

    plsc.subcore_barrier()

    hz = [None] * NCH
    hs = [None] * NCH
    for c in range(NCH):
        dst = out_hbm.at[pl.ds(row0k + c * ZW, ZW)]
        hz[c] = pltpu.async_copy(zeros_sh, dst, semz)
        if c >= WIN:
            p = c - WIN
            hz[p].wait()
            hs[p] = pltpu.async_copy(ones_v, out_hbm.at[idx2d.at[p]], sems)
    for p in range(NCH - WIN, NCH):
        hz[p].wait()
        hs[p] = pltpu.async_copy(ones_v, out_hbm.at[idx2d.at[p]], sems)
    for p in range(NCH):
        hs[p].wait()


@jax.jit
def _onehot_sc(seq_flat):
    kern = functools.partial(
        pl.kernel,
        mesh=plsc.VectorSubcoreMesh(core_axis_name="c", subcore_axis_name="s"),
        out_type=jax.ShapeDtypeStruct((ROWS * CTX,), jnp.float32),
        scratch_types=[
            pltpu.VMEM((RPW,), jnp.int32),
            pltpu.VMEM((NCH, ZCH), jnp.int32),
            pltpu.VMEM((SLICE + L * 16,), jnp.float32),
            pltpu.VMEM((ZCH,), jnp.float32),
            pltpu.VMEM_SHARED((ZW,), jnp.float32),
            pltpu.SemaphoreType.DMA,
            pltpu.SemaphoreType.DMA,
        ],
        compiler_params=pltpu.CompilerParams(needs_layout_passes=False),
    )(_body)
    return kern(seq_flat)


def kernel(sequence):
    seq_flat = sequence.reshape(ROWS).astype(jnp.int32)
    out = _onehot_sc(seq_flat)
    return out.reshape(B, S, CTX)
